# Initial kernel scaffold; baseline (speedup 1.0000x reference)
import jax
import jax.numpy as jnp
from jax.experimental import pallas as pl


def kernel(z, edge_index):
    raise NotImplementedError("write your pallas kernel here")


# speedup vs baseline: 3.1844x; 3.1844x over previous
import functools

import jax
import jax.numpy as jnp
from jax import lax
from jax.experimental import pallas as pl
from jax.experimental.pallas import tpu as pltpu
from jax.experimental.pallas import tpu_sc as plsc

_NUM_CORES = 2
_NUM_SUBCORES = 16
_NUM_WORKERS = _NUM_CORES * _NUM_SUBCORES
_LANES = 16

_N_EDGES = 320000
_D = 128
_B = 80
_EPW = _N_EDGES // _NUM_WORKERS
_NBLK = _EPW // _B


def _sc_body(z_hbm, src_hbm, dst_hbm, out_hbm,
             src_idx, dst_idx, src_rows, dst_rows, acc_mat, out_v,
             sem_s, sem_d):
  wid = lax.axis_index("s") * _NUM_CORES + lax.axis_index("c")
  base = wid * _EPW

  def blk_body(b, carry):
    off = base + b * _B
    pltpu.sync_copy(src_hbm.at[pl.ds(off, _B)], src_idx)
    pltpu.sync_copy(dst_hbm.at[pl.ds(off, _B)], dst_idx)
    cp_s = pltpu.async_copy(z_hbm.at[src_idx], src_rows, sem_s)
    cp_d = pltpu.async_copy(z_hbm.at[dst_idx], dst_rows, sem_d)
    cp_s.wait()
    cp_d.wait()

    def grp_body(g, carry2):
      e0 = g * _LANES
      for j in range(_LANES):
        acc = src_rows[e0 + j, pl.ds(0, _LANES)] * dst_rows[e0 + j, pl.ds(0, _LANES)]
        for k in range(1, _D // _LANES):
          acc = acc + (src_rows[e0 + j, pl.ds(k * _LANES, _LANES)]
                       * dst_rows[e0 + j, pl.ds(k * _LANES, _LANES)])
        acc_mat[j] = acc
      rows_iota = lax.iota(jnp.int32, _LANES)
      val = plsc.load_gather(acc_mat, [rows_iota, jnp.zeros((_LANES,), jnp.int32)])
      for l in range(1, _LANES):
        val = val + plsc.load_gather(
            acc_mat, [rows_iota, jnp.full((_LANES,), l, jnp.int32)])
      out_v[pl.ds(e0, _LANES)] = 1.0 / (1.0 + jnp.exp(-val))
      return carry2

    lax.fori_loop(0, _B // _LANES, grp_body, 0)
    pltpu.sync_copy(out_v, out_hbm.at[pl.ds(off, _B)])
    return carry

  lax.fori_loop(0, _NBLK, blk_body, 0)


@jax.jit
def _decode(z, src, dst):
  mesh = plsc.VectorSubcoreMesh(
      core_axis_name="c", subcore_axis_name="s",
      num_cores=_NUM_CORES, num_subcores=_NUM_SUBCORES)
  return pl.kernel(
      _sc_body,
      out_type=jax.ShapeDtypeStruct((_N_EDGES,), jnp.float32),
      mesh=mesh,
      compiler_params=pltpu.CompilerParams(needs_layout_passes=False),
      scratch_types=[
          pltpu.VMEM((_B,), jnp.int32),
          pltpu.VMEM((_B,), jnp.int32),
          pltpu.VMEM((_B, _D), jnp.float32),
          pltpu.VMEM((_B, _D), jnp.float32),
          pltpu.VMEM((_LANES, _LANES), jnp.float32),
          pltpu.VMEM((_B,), jnp.float32),
          pltpu.SemaphoreType.DMA,
          pltpu.SemaphoreType.DMA,
      ],
  )(z, src, dst)


def kernel(z, edge_index):
  src = edge_index[0]
  dst = edge_index[1]
  return _decode(z, src, dst)


# speedup vs baseline: 6.6403x; 2.0853x over previous
import jax
import jax.numpy as jnp
from jax import lax
from jax.experimental import pallas as pl
from jax.experimental.pallas import tpu as pltpu
from jax.experimental.pallas import tpu_sc as plsc

_NUM_CORES = 2
_NUM_SUBCORES = 16
_NUM_WORKERS = _NUM_CORES * _NUM_SUBCORES
_LANES = 16

_N_EDGES = 320000
_D = 128
_B = 80
_EPW = _N_EDGES // _NUM_WORKERS
_NBLK = _EPW // _B


def _sc_body(z_hbm, src_hbm, dst_hbm, out_hbm,
             src_idx, dst_idx, out_all,
             src_rows0, dst_rows0, src_rows1, dst_rows1, acc_mat,
             sem_s0, sem_d0, sem_s1, sem_d1):
  wid = lax.axis_index("s") * _NUM_CORES + lax.axis_index("c")
  base = wid * _EPW

  src_rows = (src_rows0, src_rows1)
  dst_rows = (dst_rows0, dst_rows1)
  sem_s = (sem_s0, sem_s1)
  sem_d = (sem_d0, sem_d1)

  pltpu.sync_copy(src_hbm.at[pl.ds(base, _EPW)], src_idx)
  pltpu.sync_copy(dst_hbm.at[pl.ds(base, _EPW)], dst_idx)

  def start(blk, p):
    off = blk * _B
    pltpu.async_copy(z_hbm.at[src_idx.at[pl.ds(off, _B)]], src_rows[p], sem_s[p])
    pltpu.async_copy(z_hbm.at[dst_idx.at[pl.ds(off, _B)]], dst_rows[p], sem_d[p])

  def wait(p):
    pltpu.make_async_copy(z_hbm.at[src_idx.at[pl.ds(0, _B)]],
                          src_rows[p], sem_s[p]).wait()
    pltpu.make_async_copy(z_hbm.at[dst_idx.at[pl.ds(0, _B)]],
                          dst_rows[p], sem_d[p]).wait()

  def compute(blk, p):
    s_rows = src_rows[p]
    d_rows = dst_rows[p]

    def grp_body(g, carry2):
      e0 = g * _LANES
      for j in range(_LANES):
        acc = s_rows[e0 + j, pl.ds(0, _LANES)] * d_rows[e0 + j, pl.ds(0, _LANES)]
        for k in range(1, _D // _LANES):
          acc = acc + (s_rows[e0 + j, pl.ds(k * _LANES, _LANES)]
                       * d_rows[e0 + j, pl.ds(k * _LANES, _LANES)])
        acc_mat[j] = acc
      rows_iota = lax.iota(jnp.int32, _LANES)
      val = plsc.load_gather(acc_mat, [rows_iota, jnp.zeros((_LANES,), jnp.int32)])
      for l in range(1, _LANES):
        val = val + plsc.load_gather(
            acc_mat, [rows_iota, jnp.full((_LANES,), l, jnp.int32)])
      out_all[pl.ds(blk * _B + e0, _LANES)] = 1.0 / (1.0 + jnp.exp(-val))
      return carry2

    lax.fori_loop(0, _B // _LANES, grp_body, 0)

  start(0, 0)

  def it_body(i, carry):
    for p in range(2):
      blk = 2 * i + p

      @pl.when(blk + 1 < _NBLK)
      def _():
        start(blk + 1, 1 - p)

      @pl.when(blk < _NBLK)
      def _():
        wait(p)
        compute(blk, p)
    return carry

  lax.fori_loop(0, (_NBLK + 1) // 2, it_body, 0)

  pltpu.sync_copy(out_all, out_hbm.at[pl.ds(base, _EPW)])


@jax.jit
def _decode(z, src, dst):
  mesh = plsc.VectorSubcoreMesh(
      core_axis_name="c", subcore_axis_name="s",
      num_cores=_NUM_CORES, num_subcores=_NUM_SUBCORES)
  return pl.kernel(
      _sc_body,
      out_type=jax.ShapeDtypeStruct((_N_EDGES,), jnp.float32),
      mesh=mesh,
      compiler_params=pltpu.CompilerParams(needs_layout_passes=False),
      scratch_types=[
          pltpu.VMEM((_EPW,), jnp.int32),
          pltpu.VMEM((_EPW,), jnp.int32),
          pltpu.VMEM((_EPW,), jnp.float32),
          pltpu.VMEM((_B, _D), jnp.float32),
          pltpu.VMEM((_B, _D), jnp.float32),
          pltpu.VMEM((_B, _D), jnp.float32),
          pltpu.VMEM((_B, _D), jnp.float32),
          pltpu.VMEM((_LANES, _LANES), jnp.float32),
          pltpu.SemaphoreType.DMA,
          pltpu.SemaphoreType.DMA,
          pltpu.SemaphoreType.DMA,
          pltpu.SemaphoreType.DMA,
      ],
  )(z, src, dst)


def kernel(z, edge_index):
  src = edge_index[0]
  dst = edge_index[1]
  return _decode(z, src, dst)
